# baseline (speedup 1.0000x reference)
import functools

import jax
import jax.numpy as jnp
import numpy as np
from jax import lax
from jax.experimental import pallas as pl
from jax.experimental.pallas import tpu as pltpu
from jax.experimental.pallas import tpu_sc as plsc

N_PTS = 1000000
RES = 256
N_VOX = RES * RES * RES

NC = 2
NS = 16
NW = NC * NS
LANES = 16

N_CHUNKS = 4
PW = 31296
P_TOT = NW * PW
CHUNK = PW // N_CHUNKS

EPS = np.float32(1e-5)
HI = np.float32(1.0) - EPS
LO = EPS


def _idx_body(x, y, z, out, xb0, xb1, yb0, yb1, zb0, zb1, wb0, wb1,
              isem, osem):
    wid = lax.axis_index("s") * NC + lax.axis_index("c")
    base = wid * PW
    xbs, ybs, zbs = (xb0, xb1), (yb0, yb1), (zb0, zb1)
    wbs = (wb0, wb1)

    def dma_in(ci, b):
        off = base + ci * CHUNK
        return (
            pltpu.async_copy(x.at[pl.ds(off, CHUNK)], xbs[b], isem.at[b]),
            pltpu.async_copy(y.at[pl.ds(off, CHUNK)], ybs[b], isem.at[b]),
            pltpu.async_copy(z.at[pl.ds(off, CHUNK)], zbs[b], isem.at[b]),
        )

    def compute(b):
        xb, yb, zb, wb = xbs[b], ybs[b], zbs[b], wbs[b]

        def idx_body(i, _):
            s = i * LANES
            xv = xb[pl.ds(s, LANES)]
            yv = yb[pl.ds(s, LANES)]
            zv = zb[pl.ds(s, LANES)]
            ix = (xv * np.float32(RES)).astype(jnp.int32)
            iy = (yv * np.float32(RES)).astype(jnp.int32)
            iz = (zv * np.float32(RES)).astype(jnp.int32)
            cmin = jnp.minimum(jnp.minimum(xv, yv), zv)
            cmax = jnp.maximum(jnp.maximum(xv, yv), zv)
            inv = (cmax >= HI) | (cmin < LO)
            lin = (ix * RES + iy) * RES + iz
            wb[pl.ds(s, LANES)] = jnp.where(inv, N_VOX, lin)
            return 0

        lax.fori_loop(0, CHUNK // LANES, idx_body, 0)

    in_dmas = {0: dma_in(0, 0)}
    o_dmas = {}
    for ci in range(N_CHUNKS):
        b = ci & 1
        if ci + 1 < N_CHUNKS:
            in_dmas[ci + 1] = dma_in(ci + 1, 1 - b)
        for d in in_dmas.pop(ci):
            d.wait()
        if ci >= 2:
            o_dmas.pop(ci - 2).wait()
        compute(b)
        o_dmas[ci] = pltpu.async_copy(
            wbs[b], out.at[pl.ds(base + ci * CHUNK, CHUNK)], osem.at[b]
        )
    o_dmas.pop(N_CHUNKS - 2).wait()
    o_dmas.pop(N_CHUNKS - 1).wait()


def _gat_body(idx, gw, out, ib0, ib1, gb0, gb1, isem, gsem, osem):
    wid = lax.axis_index("s") * NC + lax.axis_index("c")
    base = wid * PW
    ibs, gbs = (ib0, ib1), (gb0, gb1)

    def dma_in(ci, b):
        off = base + ci * CHUNK
        return pltpu.async_copy(idx.at[pl.ds(off, CHUNK)], ibs[b], isem.at[b])

    in_dmas = {0: dma_in(0, 0)}
    o_dmas = {}
    for ci in range(N_CHUNKS):
        b = ci & 1
        if ci + 1 < N_CHUNKS:
            in_dmas[ci + 1] = dma_in(ci + 1, 1 - b)
        in_dmas.pop(ci).wait()
        if ci >= 2:
            o_dmas.pop(ci - 2).wait()
        pltpu.async_copy(gw.at[ibs[b]], gbs[b], gsem.at[b]).wait()
        o_dmas[ci] = pltpu.async_copy(
            gbs[b], out.at[pl.ds(base + ci * CHUNK, CHUNK)], osem.at[b]
        )
    o_dmas.pop(N_CHUNKS - 2).wait()
    o_dmas.pop(N_CHUNKS - 1).wait()


@jax.jit
def kernel(pts, grid_flat):
    p = jnp.pad(pts, ((0, P_TOT - N_PTS), (0, 0)))
    x, y, z = p[:, 0], p[:, 1], p[:, 2]
    gw = grid_flat.astype(jnp.int32)

    mesh = plsc.VectorSubcoreMesh(
        core_axis_name="c", subcore_axis_name="s", num_cores=NC, num_subcores=NS
    )
    k1 = functools.partial(
        pl.kernel,
        mesh=mesh,
        out_type=jax.ShapeDtypeStruct((P_TOT,), jnp.int32),
        scratch_types=(
            [pltpu.VMEM((CHUNK,), jnp.float32)] * 6
            + [pltpu.VMEM((CHUNK,), jnp.int32)] * 2
            + [pltpu.SemaphoreType.DMA((2,))] * 2
        ),
    )(_idx_body)
    k2 = functools.partial(
        pl.kernel,
        mesh=mesh,
        out_type=jax.ShapeDtypeStruct((P_TOT,), jnp.int32),
        scratch_types=(
            [pltpu.VMEM((CHUNK,), jnp.int32)] * 4
            + [pltpu.SemaphoreType.DMA((2,))] * 3
        ),
    )(_gat_body)
    widx = k1(x, y, z)
    vals = k2(widx, gw)
    return vals[:N_PTS].astype(jnp.bool_)


# speedup vs baseline: 1.0051x; 1.0051x over previous
import functools

import jax
import jax.numpy as jnp
import numpy as np
from jax import lax
from jax.experimental import pallas as pl
from jax.experimental.pallas import tpu as pltpu
from jax.experimental.pallas import tpu_sc as plsc

N_PTS = 1000000
RES = 256
N_VOX = RES * RES * RES

NC = 2
NS = 16
NW = NC * NS
LANES = 16

N_CHUNKS = 6
PW = 31296
P_TOT = NW * PW
CHUNK = PW // N_CHUNKS

EPS = np.float32(1e-5)
HI = np.float32(1.0) - EPS
LO = EPS


def _body(x, y, z, gw, out,
          xb0, xb1, yb0, yb1, zb0, zb1, wb0, wb1, gb0, gb1,
          isem, gsem, osem):
    wid = lax.axis_index("s") * NC + lax.axis_index("c")
    base = wid * PW
    xbs, ybs, zbs = (xb0, xb1), (yb0, yb1), (zb0, zb1)
    wbs, gbs = (wb0, wb1), (gb0, gb1)

    def dma_in(ci, b):
        off = base + ci * CHUNK
        return (
            pltpu.async_copy(x.at[pl.ds(off, CHUNK)], xbs[b], isem.at[b]),
            pltpu.async_copy(y.at[pl.ds(off, CHUNK)], ybs[b], isem.at[b]),
            pltpu.async_copy(z.at[pl.ds(off, CHUNK)], zbs[b], isem.at[b]),
        )

    def compute(b):
        xb, yb, zb, wb = xbs[b], ybs[b], zbs[b], wbs[b]

        def idx_body(i, _):
            s = i * LANES
            xv = xb[pl.ds(s, LANES)]
            yv = yb[pl.ds(s, LANES)]
            zv = zb[pl.ds(s, LANES)]
            ix = (xv * np.float32(RES)).astype(jnp.int32)
            iy = (yv * np.float32(RES)).astype(jnp.int32)
            iz = (zv * np.float32(RES)).astype(jnp.int32)
            cmin = jnp.minimum(jnp.minimum(xv, yv), zv)
            cmax = jnp.maximum(jnp.maximum(xv, yv), zv)
            inv = (cmax >= HI) | (cmin < LO)
            lin = (ix * RES + iy) * RES + iz
            wb[pl.ds(s, LANES)] = jnp.where(inv, N_VOX, lin)
            return 0

        lax.fori_loop(0, CHUNK // LANES, idx_body, 0)

    in_dmas = {0: dma_in(0, 0)}
    g_dmas = {}
    o_dmas = {}
    for ci in range(N_CHUNKS):
        b = ci & 1
        if ci + 1 < N_CHUNKS:
            in_dmas[ci + 1] = dma_in(ci + 1, 1 - b)
        for d in in_dmas.pop(ci):
            d.wait()
        compute(b)
        if ci >= 1:
            g_dmas.pop(ci - 1).wait()
            o_dmas[ci - 1] = pltpu.async_copy(
                gbs[1 - b],
                out.at[pl.ds(base + (ci - 1) * CHUNK, CHUNK)],
                osem.at[1 - b],
            )
        if ci >= 2:
            o_dmas.pop(ci - 2).wait()
        g_dmas[ci] = pltpu.async_copy(gw.at[wbs[b]], gbs[b], gsem.at[b])

    last = N_CHUNKS - 1
    b = last & 1
    g_dmas.pop(last).wait()
    pltpu.async_copy(
        gbs[b], out.at[pl.ds(base + last * CHUNK, CHUNK)], osem.at[b]
    ).wait()
    o_dmas.pop(last - 1).wait()


@jax.jit
def kernel(pts, grid_flat):
    p = jnp.pad(pts, ((0, P_TOT - N_PTS), (0, 0)))
    x, y, z = p[:, 0], p[:, 1], p[:, 2]
    gw = grid_flat.astype(jnp.int32)

    mesh = plsc.VectorSubcoreMesh(
        core_axis_name="c", subcore_axis_name="s", num_cores=NC, num_subcores=NS
    )
    run = functools.partial(
        pl.kernel,
        mesh=mesh,
        out_type=jax.ShapeDtypeStruct((P_TOT,), jnp.int32),
        scratch_types=(
            [pltpu.VMEM((CHUNK,), jnp.float32)] * 6
            + [pltpu.VMEM((CHUNK,), jnp.int32)] * 4
            + [pltpu.SemaphoreType.DMA((2,))] * 3
        ),
    )(_body)
    out = run(x, y, z, gw)
    return out[:N_PTS].astype(jnp.bool_)


# speedup vs baseline: 1.0067x; 1.0015x over previous
import functools

import jax
import jax.numpy as jnp
import numpy as np
from jax import lax
from jax.experimental import pallas as pl
from jax.experimental.pallas import tpu as pltpu
from jax.experimental.pallas import tpu_sc as plsc

N_PTS = 1000000
RES = 256
N_VOX = RES * RES * RES

NC = 2
NS = 16
NW = NC * NS
LANES = 16

N_CHUNKS = 4
PW = 31296
P_TOT = NW * PW
CHUNK = PW // N_CHUNKS

EPS = np.float32(1e-5)
HI = np.float32(1.0) - EPS
LO = EPS


def _body(x, y, z, gw, out,
          xb0, xb1, yb0, yb1, zb0, zb1, wb0, wb1, gb0, gb1,
          isem, gsem, osem):
    wid = lax.axis_index("s") * NC + lax.axis_index("c")
    base = wid * PW
    xbs, ybs, zbs = (xb0, xb1), (yb0, yb1), (zb0, zb1)
    wbs, gbs = (wb0, wb1), (gb0, gb1)

    def dma_in(ci, b):
        off = base + ci * CHUNK
        return (
            pltpu.async_copy(x.at[pl.ds(off, CHUNK)], xbs[b], isem.at[b]),
            pltpu.async_copy(y.at[pl.ds(off, CHUNK)], ybs[b], isem.at[b]),
            pltpu.async_copy(z.at[pl.ds(off, CHUNK)], zbs[b], isem.at[b]),
        )

    def compute(b):
        xb, yb, zb, wb = xbs[b], ybs[b], zbs[b], wbs[b]

        def idx_body(i, _):
            s = i * LANES
            xv = xb[pl.ds(s, LANES)]
            yv = yb[pl.ds(s, LANES)]
            zv = zb[pl.ds(s, LANES)]
            ix = (xv * np.float32(RES)).astype(jnp.int32)
            iy = (yv * np.float32(RES)).astype(jnp.int32)
            iz = (zv * np.float32(RES)).astype(jnp.int32)
            cmin = jnp.minimum(jnp.minimum(xv, yv), zv)
            cmax = jnp.maximum(jnp.maximum(xv, yv), zv)
            inv = (cmax >= HI) | (cmin < LO)
            lin = (ix * RES + iy) * RES + iz
            wb[pl.ds(s, LANES)] = jnp.where(inv, N_VOX, lin)
            return 0

        lax.fori_loop(0, CHUNK // LANES, idx_body, 0)

    in_dmas = {0: dma_in(0, 0)}
    g_dmas = {}
    o_dmas = {}
    for ci in range(N_CHUNKS):
        b = ci & 1
        if ci + 1 < N_CHUNKS:
            in_dmas[ci + 1] = dma_in(ci + 1, 1 - b)
        for d in in_dmas.pop(ci):
            d.wait()
        compute(b)
        if ci >= 1:
            g_dmas.pop(ci - 1).wait()
            o_dmas[ci - 1] = pltpu.async_copy(
                gbs[1 - b],
                out.at[pl.ds(base + (ci - 1) * CHUNK, CHUNK)],
                osem.at[1 - b],
            )
        if ci >= 2:
            o_dmas.pop(ci - 2).wait()
        g_dmas[ci] = pltpu.async_copy(gw.at[wbs[b]], gbs[b], gsem.at[b])

    last = N_CHUNKS - 1
    b = last & 1
    g_dmas.pop(last).wait()
    pltpu.async_copy(
        gbs[b], out.at[pl.ds(base + last * CHUNK, CHUNK)], osem.at[b]
    ).wait()
    o_dmas.pop(last - 1).wait()


@jax.jit
def kernel(pts, grid_flat):
    p = jnp.pad(pts, ((0, P_TOT - N_PTS), (0, 0)))
    x, y, z = p[:, 0], p[:, 1], p[:, 2]
    gw = grid_flat.astype(jnp.int32)

    mesh = plsc.VectorSubcoreMesh(
        core_axis_name="c", subcore_axis_name="s", num_cores=NC, num_subcores=NS
    )
    run = functools.partial(
        pl.kernel,
        mesh=mesh,
        out_type=jax.ShapeDtypeStruct((P_TOT,), jnp.int32),
        scratch_types=(
            [pltpu.VMEM((CHUNK,), jnp.float32)] * 6
            + [pltpu.VMEM((CHUNK,), jnp.int32)] * 4
            + [pltpu.SemaphoreType.DMA((2,))] * 3
        ),
    )(_body)
    out = run(x, y, z, gw)
    return out[:N_PTS].astype(jnp.bool_)
